# baseline (speedup 1.0000x reference)
import functools

import jax
import jax.numpy as jnp
import numpy as np
from jax import lax
from jax.experimental import pallas as pl
from jax.experimental.pallas import tpu as pltpu
from jax.experimental.pallas import tpu_sc as plsc

_B = 2
_T = 4096
_H = 4096
_K = 8
_SLOTS = 64
_TB = 1024
_NT = _T // _TB

_NC = 2
_NS = 16
_NW = _NC * _NS

_OIDX = np.arange(_B * _K, dtype=np.int32) + np.where(
    np.arange(_B * _K) >= _K, _SLOTS - _K, 0
).astype(np.int32)


def _gate_body(w_ref, b_ref, x_ref, scores_ref, idx_ref, acc_ref):
    bi = pl.program_id(0)
    ti = pl.program_id(1)
    x = x_ref[0]
    w = w_ref[...]
    logits = lax.dot_general(
        w, x, (((1,), (1,)), ((), ())), preferred_element_type=jnp.float32
    )
    logits = logits + b_ref[0, 0]
    scores_ref[...] = jax.nn.sigmoid(logits)[0]
    acc_ref[pl.ds(ti, 1), :] = logits

    @pl.when(ti == _NT - 1)
    def _():
        vals = acc_ref[...]
        rows = lax.broadcasted_iota(jnp.int32, (_NT, _TB), 0)
        cols = lax.broadcasted_iota(jnp.int32, (_NT, _TB), 1)
        gpos = rows * _TB + cols
        big = jnp.int32(_T)
        neg = jnp.float32(-jnp.inf)
        for j in range(_K):
            m = jnp.max(vals)
            ij = jnp.min(jnp.where(vals == m, gpos, big))
            idx_ref[bi * _K + j] = bi * _T + ij
            vals = jnp.where(gpos == ij, neg, vals)


def _gate(enc, w1h, b2d):
    return pl.pallas_call(
        _gate_body,
        grid=(_B, _NT),
        in_specs=[
            pl.BlockSpec((1, _H), lambda b, t: (0, 0)),
            pl.BlockSpec(memory_space=pltpu.SMEM),
            pl.BlockSpec((1, _TB, _H), lambda b, t: (b, t, 0)),
        ],
        out_specs=[
            pl.BlockSpec((_TB,), lambda b, t: (b * _NT + t,)),
            pl.BlockSpec(memory_space=pltpu.SMEM),
        ],
        out_shape=[
            jax.ShapeDtypeStruct((_B * _T,), jnp.float32),
            jax.ShapeDtypeStruct((_B * _K,), jnp.int32),
        ],
        scratch_shapes=[pltpu.VMEM((_NT, _TB), jnp.float32)],
    )(w1h, b2d, enc)


def _sc_gather_memory(enc2d, gidx, oidx, mem_ref):
    mesh = plsc.VectorSubcoreMesh(core_axis_name="c", subcore_axis_name="s")

    @functools.partial(
        pl.kernel,
        mesh=mesh,
        scratch_types=[
            pltpu.VMEM((_K,), jnp.int32),
            pltpu.VMEM((_K,), jnp.int32),
            pltpu.VMEM((_K, _H), jnp.float32),
            pltpu.SemaphoreType.DMA,
        ],
    )
    def k(enc_hbm, gidx_hbm, oidx_hbm, out_hbm, idx_v, oidx_v, rows_v, sem):
        cid = lax.axis_index("c")
        sid = lax.axis_index("s")
        wid = sid * _NC + cid

        for w in range(_B):

            @pl.when(wid == w)
            def _(w=w):
                c1 = pltpu.async_copy(gidx_hbm.at[pl.ds(_K * w, _K)], idx_v, sem)
                c2 = pltpu.async_copy(oidx_hbm.at[pl.ds(_K * w, _K)], oidx_v, sem)
                c1.wait()
                c2.wait()
                pltpu.async_copy(enc_hbm.at[idx_v], rows_v, sem).wait()
                pltpu.async_copy(rows_v, out_hbm.at[oidx_v], sem).wait()

    k(enc2d, gidx, oidx, mem_ref)


def _sc_zero_memory():
    mesh = plsc.VectorSubcoreMesh(core_axis_name="c", subcore_axis_name="s")

    @functools.partial(
        pl.kernel,
        mesh=mesh,
        out_type=jax.ShapeDtypeStruct((_B * _SLOTS, _H), jnp.float32),
        scratch_types=[
            pltpu.VMEM((_H,), jnp.float32),
            pltpu.SemaphoreType.DMA,
        ],
    )
    def k(out_hbm, zrow_v, sem):
        cid = lax.axis_index("c")
        sid = lax.axis_index("s")
        wid = sid * _NC + cid
        z16 = jnp.zeros((16,), jnp.float32)

        @pl.loop(0, _H, step=16)
        def _(i):
            zrow_v[pl.ds(i, 16)] = z16
        copies = [
            pltpu.async_copy(zrow_v, out_hbm.at[wid * 4 + r], sem)
            for r in range(4)
        ]
        for c in copies:
            c.wait()

    return k()


def _sc_stream_probe(enc2d):
    mesh = plsc.VectorSubcoreMesh(core_axis_name="c", subcore_axis_name="s")
    rows_per_w = (_B * _T) // _NW
    chunk = 8
    nch = rows_per_w // chunk

    @functools.partial(
        pl.kernel,
        mesh=mesh,
        out_type=jax.ShapeDtypeStruct((_NW, _H), jnp.float32),
        scratch_types=[
            pltpu.VMEM((chunk, _H), jnp.float32),
            pltpu.VMEM((chunk, _H), jnp.float32),
            pltpu.SemaphoreType.DMA,
            pltpu.SemaphoreType.DMA,
        ],
    )
    def k(enc_hbm, out_hbm, buf0, buf1, sem0, sem1):
        cid = lax.axis_index("c")
        sid = lax.axis_index("s")
        wid = sid * _NC + cid
        base = wid * rows_per_w
        bufs = (buf0, buf1)
        sems = (sem0, sem1)
        pltpu.async_copy(enc_hbm.at[pl.ds(base, chunk)], buf0, sem0)
        for c in range(1, nch):
            pltpu.async_copy(
                enc_hbm.at[pl.ds(base + c * chunk, chunk)], bufs[c % 2], sems[c % 2]
            )
            pltpu.make_async_copy(
                enc_hbm.at[pl.ds(base + (c - 1) * chunk, chunk)],
                bufs[(c - 1) % 2],
                sems[(c - 1) % 2],
            ).wait()
        pltpu.make_async_copy(
            enc_hbm.at[pl.ds(base + (nch - 1) * chunk, chunk)],
            bufs[(nch - 1) % 2],
            sems[(nch - 1) % 2],
        ).wait()
        pltpu.sync_copy(bufs[(nch - 1) % 2].at[0], out_hbm.at[wid])

    return k(enc2d)


def kernel(enc_hidden, W, b):
    w1h = W.reshape(1, _H)
    b2d = b.reshape(1, 1)
    scores_flat, gidx = _gate(enc_hidden, w1h, b2d)
    gate_scores = scores_flat.reshape(_B, _T)
    enc2d = enc_hidden.reshape(_B * _T, _H)
    chk = _sc_stream_probe(enc2d)
    gathered = jnp.take(enc2d, gidx, axis=0).reshape(_B, _K, _H)
    memory = jnp.zeros((_B, _SLOTS, _H), jnp.float32).at[:, :_K, :].set(gathered)
    memory = memory + 0.0 * chk[0, 0]
    return (memory, gate_scores)


# speedup vs baseline: 1.6002x; 1.6002x over previous
import functools

import jax
import jax.numpy as jnp
from jax import lax
from jax.experimental import pallas as pl
from jax.experimental.pallas import tpu as pltpu
from jax.experimental.pallas import tpu_sc as plsc

_B = 2
_T = 4096
_H = 4096
_K = 8
_SLOTS = 64
_TB = 1024
_NT = _T // _TB

_NC = 2
_NS = 16
_NW = _NC * _NS


def _gate_body(w_ref, b_ref, x_ref, scores_ref, idx_ref, acc_ref):
    ti = pl.program_id(0)
    bi = pl.program_id(1)
    x = x_ref[0]
    w = w_ref[...]
    logits = lax.dot_general(
        w, x, (((1,), (1,)), ((), ())), preferred_element_type=jnp.float32
    )
    logits = logits + b_ref[0, 0]
    scores_ref[pl.ds(bi, 1), :] = jax.nn.sigmoid(logits)
    acc_ref[pl.ds(bi, 1), pl.ds(ti, 1), :] = logits[None]

    @pl.when(ti == _NT - 1)
    def _():
        vals = acc_ref[bi]
        rows = lax.broadcasted_iota(jnp.int32, (_NT, _TB), 0)
        cols = lax.broadcasted_iota(jnp.int32, (_NT, _TB), 1)
        gpos = rows * _TB + cols
        big = jnp.int32(_T)
        neg = jnp.float32(-jnp.inf)
        for j in range(_K):
            m = jnp.max(vals)
            ij = jnp.min(jnp.where(vals == m, gpos, big))
            idx_ref[bi * _K + j] = bi * _T + ij
            vals = jnp.where(gpos == ij, neg, vals)


def _gate(enc, w1h, b2d):
    return pl.pallas_call(
        _gate_body,
        grid=(_NT, _B),
        in_specs=[
            pl.BlockSpec((1, _H), lambda t, b: (0, 0)),
            pl.BlockSpec(memory_space=pltpu.SMEM),
            pl.BlockSpec((1, _TB, _H), lambda t, b: (b, t, 0)),
        ],
        out_specs=[
            pl.BlockSpec((_B, _TB), lambda t, b: (0, t)),
            pl.BlockSpec(memory_space=pltpu.SMEM),
        ],
        out_shape=[
            jax.ShapeDtypeStruct((_B, _T), jnp.float32),
            jax.ShapeDtypeStruct((_B * _K,), jnp.int32),
        ],
        scratch_shapes=[pltpu.VMEM((_B, _NT, _TB), jnp.float32)],
    )(w1h, b2d, enc)


def _sc_zero_memory():
    mesh = plsc.VectorSubcoreMesh(core_axis_name="c", subcore_axis_name="s")

    @functools.partial(
        pl.kernel,
        mesh=mesh,
        out_type=jax.ShapeDtypeStruct((_B * _SLOTS, _H), jnp.float32),
        scratch_types=[
            pltpu.VMEM((_H,), jnp.float32),
            pltpu.SemaphoreType.DMA,
        ],
    )
    def k(out_hbm, zrow_v, sem):
        cid = lax.axis_index("c")
        sid = lax.axis_index("s")
        wid = sid * _NC + cid
        z16 = jnp.zeros((16,), jnp.float32)

        @pl.loop(0, _H, step=16)
        def _(i):
            zrow_v[pl.ds(i, 16)] = z16

        copies = [
            pltpu.async_copy(zrow_v, out_hbm.at[wid * 4 + r], sem)
            for r in range(4)
        ]
        for c in copies:
            c.wait()

    return k()


def _sc_gather_memory(enc2d, gidx, mem_ref):
    mesh = plsc.VectorSubcoreMesh(core_axis_name="c", subcore_axis_name="s")

    @functools.partial(
        pl.kernel,
        mesh=mesh,
        scratch_types=[
            pltpu.VMEM((_K,), jnp.int32),
            pltpu.VMEM((_K, _H), jnp.float32),
            pltpu.SemaphoreType.DMA,
        ],
    )
    def k(enc_hbm, gidx_hbm, out_hbm, idx_v, rows_v, sem):
        cid = lax.axis_index("c")
        sid = lax.axis_index("s")
        wid = sid * _NC + cid

        for w in range(_B):

            @pl.when(wid == w)
            def _(w=w):
                pltpu.sync_copy(gidx_hbm.at[pl.ds(_K * w, _K)], idx_v)
                pltpu.async_copy(enc_hbm.at[idx_v], rows_v, sem).wait()
                pltpu.sync_copy(rows_v, out_hbm.at[pl.ds(_SLOTS * w, _K)])

    k(enc2d, gidx, mem_ref)


def kernel(enc_hidden, W, b):
    w1h = W.reshape(1, _H)
    b2d = b.reshape(1, 1)
    gate_scores, gidx = _gate(enc_hidden, w1h, b2d)
    enc2d = enc_hidden.reshape(_B * _T, _H)
    mem0 = _sc_zero_memory()
    mem_ref = jax.new_ref(mem0)
    _sc_gather_memory(enc2d, gidx, mem_ref)
    memory = jax.freeze(mem_ref).reshape(_B, _SLOTS, _H)
    return (memory, gate_scores)


# speedup vs baseline: 1.6368x; 1.0228x over previous
import functools

import jax
import jax.numpy as jnp
from jax import lax
from jax.experimental import pallas as pl
from jax.experimental.pallas import tpu as pltpu
from jax.experimental.pallas import tpu_sc as plsc

_B = 2
_T = 4096
_H = 4096
_K = 8
_SLOTS = 64
_TB = 1024
_NT = _T // _TB

_NC = 2
_NS = 16
_NW = _NC * _NS


def _gate_body(w_ref, b_ref, x_ref, scores_ref, idx_ref, acc_ref):
    ti = pl.program_id(0)
    bi = pl.program_id(1)
    x = x_ref[0]
    w = w_ref[...]
    logits = lax.dot_general(
        w, x, (((1,), (1,)), ((), ())), preferred_element_type=jnp.float32
    )
    logits = logits + b_ref[0, 0]
    scores_ref[pl.ds(bi, 1), :] = jax.nn.sigmoid(logits)
    acc_ref[pl.ds(bi, 1), pl.ds(ti, 1), :] = logits[None]

    @pl.when(ti == _NT - 1)
    def _():
        vals = acc_ref[bi]
        rows = lax.broadcasted_iota(jnp.int32, (_NT, _TB), 0)
        cols = lax.broadcasted_iota(jnp.int32, (_NT, _TB), 1)
        gpos = rows * _TB + cols
        big = jnp.int32(_T)
        neg = jnp.float32(-jnp.inf)
        for j in range(_K):
            m = jnp.max(vals)
            ij = jnp.min(jnp.where(vals == m, gpos, big))
            idx_ref[(bi * _K + j) * 8] = bi * _T + ij
            vals = jnp.where(gpos == ij, neg, vals)


def _gate(enc, w1h, b2d):
    return pl.pallas_call(
        _gate_body,
        grid=(_NT, _B),
        in_specs=[
            pl.BlockSpec((1, _H), lambda t, b: (0, 0)),
            pl.BlockSpec(memory_space=pltpu.SMEM),
            pl.BlockSpec((1, _TB, _H), lambda t, b: (b, t, 0)),
        ],
        out_specs=[
            pl.BlockSpec((_B, _TB), lambda t, b: (0, t)),
            pl.BlockSpec(memory_space=pltpu.SMEM),
        ],
        out_shape=[
            jax.ShapeDtypeStruct((_B, _T), jnp.float32),
            jax.ShapeDtypeStruct((_B * _K * 8,), jnp.int32),
        ],
        scratch_shapes=[pltpu.VMEM((_B, _NT, _TB), jnp.float32)],
    )(w1h, b2d, enc)


def _sc_zero_memory():
    mesh = plsc.VectorSubcoreMesh(core_axis_name="c", subcore_axis_name="s")

    @functools.partial(
        pl.kernel,
        mesh=mesh,
        out_type=jax.ShapeDtypeStruct((_B * _SLOTS, _H), jnp.float32),
        scratch_types=[
            pltpu.VMEM((_H,), jnp.float32),
            pltpu.SemaphoreType.DMA,
        ],
    )
    def k(out_hbm, zrow_v, sem):
        cid = lax.axis_index("c")
        sid = lax.axis_index("s")
        wid = sid * _NC + cid
        z16 = jnp.zeros((16,), jnp.float32)

        @pl.loop(0, _H, step=16)
        def _(i):
            zrow_v[pl.ds(i, 16)] = z16

        copies = [
            pltpu.async_copy(zrow_v, out_hbm.at[wid * 4 + r], sem)
            for r in range(4)
        ]
        for c in copies:
            c.wait()

    return k()


def _sc_gather_memory(enc2d, gidx, mem_ref):
    mesh = plsc.VectorSubcoreMesh(core_axis_name="c", subcore_axis_name="s")

    @functools.partial(
        pl.kernel,
        mesh=mesh,
        scratch_types=[
            pltpu.VMEM((_B * _K * 8,), jnp.int32),
            pltpu.VMEM((1, _H), jnp.float32),
            pltpu.SemaphoreType.DMA,
        ],
    )
    def k(enc_hbm, gidx_hbm, out_hbm, idx_v, row_v, sem):
        cid = lax.axis_index("c")
        sid = lax.axis_index("s")
        wid = sid * _NC + cid

        @pl.when(wid < _B * _K)
        def _():
            pltpu.sync_copy(gidx_hbm, idx_v)
            pltpu.async_copy(
                enc_hbm.at[idx_v.at[pl.ds(wid * 8, 1)]], row_v, sem
            ).wait()
            dst = (wid // _K) * _SLOTS + lax.rem(wid, _K)
            pltpu.sync_copy(row_v, out_hbm.at[pl.ds(dst, 1)])

    k(enc2d, gidx, mem_ref)


def kernel(enc_hidden, W, b):
    w1h = W.reshape(1, _H)
    b2d = b.reshape(1, 1)
    gate_scores, gidx = _gate(enc_hidden, w1h, b2d)
    enc2d = enc_hidden.reshape(_B * _T, _H)
    mem0 = _sc_zero_memory()
    mem_ref = jax.new_ref(mem0)
    _sc_gather_memory(enc2d, gidx, mem_ref)
    memory = jax.freeze(mem_ref).reshape(_B, _SLOTS, _H)
    return (memory, gate_scores)
